# Initial kernel scaffold; baseline (speedup 1.0000x reference)
import jax
import jax.numpy as jnp
from jax.experimental import pallas as pl


def kernel(x_scalar, vec, w_angle_q, w_dih_q, b_q, w_angle_k, w_dih_k, b_k):
    raise NotImplementedError("write your pallas kernel here")


# speedup vs baseline: 13.7889x; 13.7889x over previous
import functools

import jax
import jax.numpy as jnp
from jax import lax
from jax.experimental import pallas as pl
from jax.experimental.pallas import tpu as pltpu
from jax.experimental.pallas import tpu_sc as plsc

B, N, H, W = 2, 2048, 128, 3
EPS = 1e-8
EPS2 = EPS * EPS
C = 64
NW = 32
LANES = 16
NCG = H // LANES
OFFS = (-3, -2, -1, 1, 2, 3)
VROWS = C + 16
UROWS = C + 6
XROWS = C + 16


def _rsqrt_nr(s):
    i = lax.bitcast_convert_type(s, jnp.int32)
    y = lax.bitcast_convert_type(jnp.int32(0x5F3759DF) - (i >> 1), jnp.float32)
    for _ in range(3):
        y = y * (1.5 - 0.5 * s * y * y)
    return y


def _inv_norm(s):
    return jnp.where(s > EPS2, _rsqrt_nr(s), 1.0 / EPS)


def _sc_body(vec_hbm, x_hbm, w_hbm, q_hbm, k_hbm, vecl, xl, ul, ql, kl, wl):
    wid = lax.axis_index("s") * 2 + lax.axis_index("c")
    n0 = wid * C
    sv = jnp.clip(n0 - 8, 0, N - VROWS)
    sx = jnp.clip(n0 - 8, 0, N - XROWS)

    pltpu.sync_copy(w_hbm, wl)

    def chunk_body(it, _):
        bb = it * N
        pltpu.sync_copy(
            vec_hbm.at[pl.ds(pl.multiple_of(3 * (bb + sv), 8), 3 * VROWS)],
            vecl)
        pltpu.sync_copy(
            x_hbm.at[pl.ds(pl.multiple_of(bb + sx, 8), XROWS)], xl)

        def phase_a(ii, _):
            n = n0 - 3 + ii
            r = jnp.clip(n - sv, 0, VROWS - 1)
            vi_ok = jnp.where((n >= 0) & (n < N), 1.0, 0.0)

            def ch_a(c, _):
                cs = c * LANES
                sl = pl.ds(cs, LANES)
                vix = vecl[3 * r, sl]
                viy = vecl[3 * r + 1, sl]
                viz = vecl[3 * r + 2, sl]
                ux = jnp.zeros((LANES,), jnp.float32)
                uy = jnp.zeros((LANES,), jnp.float32)
                uz = jnp.zeros((LANES,), jnp.float32)
                for o in OFFS:
                    n2 = n + o
                    r2 = jnp.clip(n2 - sv, 0, VROWS - 1)
                    bx = vecl[3 * r2, sl] - vix
                    by = vecl[3 * r2 + 1, sl] - viy
                    bz = vecl[3 * r2 + 2, sl] - viz
                    s = bx * bx + by * by + bz * bz
                    ok = vi_ok * jnp.where((n2 >= 0) & (n2 < N), 1.0, 0.0)
                    f = _inv_norm(s) * ok
                    ux = ux + bx * f
                    uy = uy + by * f
                    uz = uz + bz * f
                ul[3 * ii, sl] = ux
                ul[3 * ii + 1, sl] = uy
                ul[3 * ii + 2, sl] = uz
                return 0

            lax.fori_loop(0, NCG, ch_a, 0, unroll=False)
            return 0

        lax.fori_loop(0, UROWS, phase_a, 0, unroll=False)

        def phase_b(c, _):
            cs = c * LANES
            sl = pl.ds(cs, LANES)
            w0q = wl[0, sl]
            w1q = wl[1, sl]
            w2q = wl[2, sl]
            w0k = wl[3, sl]
            w1k = wl[4, sl]
            w2k = wl[5, sl]

            def node_b(i, _):
                n = n0 + i
                r = n - sv
                vix = vecl[3 * r, sl]
                viy = vecl[3 * r + 1, sl]
                viz = vecl[3 * r + 2, sl]
                ur = i + 3
                uix = ul[3 * ur, sl]
                uiy = ul[3 * ur + 1, sl]
                uiz = ul[3 * ur + 2, sl]
                s_ui = uix * uix + uiy * uiy + uiz * uiz
                inv_ui = _inv_norm(s_ui)
                q_acc = jnp.zeros((LANES,), jnp.float32)
                k_acc = jnp.zeros((LANES,), jnp.float32)
                for o in OFFS:
                    n2 = n + o
                    ok = jnp.where((n2 >= 0) & (n2 < N), 1.0, 0.0)
                    r2 = jnp.clip(n2 - sv, 0, VROWS - 1)
                    bx = vecl[3 * r2, sl] - vix
                    by = vecl[3 * r2 + 1, sl] - viy
                    bz = vecl[3 * r2 + 2, sl] - viz
                    s_e = bx * bx + by * by + bz * bz
                    inv_e = _inv_norm(s_e)
                    ex = bx * inv_e
                    ey = by * inv_e
                    ez = bz * inv_e
                    u2 = ur + o
                    ujx = ul[3 * u2, sl]
                    ujy = ul[3 * u2 + 1, sl]
                    ujz = ul[3 * u2 + 2, sl]
                    d_i = uix * ex + uiy * ey + uiz * ez
                    d_j = ujx * ex + ujy * ey + ujz * ez
                    ang = jnp.maximum(jnp.minimum(d_i * inv_ui, 1.0), -1.0)
                    uipx = uix - d_i * ex
                    uipy = uiy - d_i * ey
                    uipz = uiz - d_i * ez
                    ujpx = ujx - d_j * ex
                    ujpy = ujy - d_j * ey
                    ujpz = ujz - d_j * ez
                    s_pi = uipx * uipx + uipy * uipy + uipz * uipz
                    s_pj = ujpx * ujpx + ujpy * ujpy + ujpz * ujpz
                    dotp = uipx * ujpx + uipy * ujpy + uipz * ujpz
                    dih = dotp * _inv_norm(s_pi) * _inv_norm(s_pj)
                    dih = jnp.maximum(jnp.minimum(dih, 1.0), -1.0)
                    zq = w2q + ang * w0q + dih * w1q
                    zk = w2k + ang * w0k + dih * w1k
                    gq = 1.0 / (1.0 + jnp.exp(-zq))
                    gk = 1.0 / (1.0 + jnp.exp(-zk))
                    rx = jnp.clip(n2 - sx, 0, XROWS - 1)
                    xj = xl[rx, sl] * ok
                    q_acc = q_acc + gq * xj
                    k_acc = k_acc + gk * xj
                ql[i, sl] = q_acc
                kl[i, sl] = k_acc
                return 0

            lax.fori_loop(0, C, node_b, 0, unroll=False)
            return 0

        lax.fori_loop(0, NCG, phase_b, 0, unroll=False)

        pltpu.sync_copy(ql, q_hbm.at[pl.ds(pl.multiple_of(bb + n0, 8), C)])
        pltpu.sync_copy(kl, k_hbm.at[pl.ds(pl.multiple_of(bb + n0, 8), C)])
        return 0

    lax.fori_loop(0, B, chunk_body, 0, unroll=False)


@jax.jit
def kernel(x_scalar, vec, w_angle_q, w_dih_q, b_q, w_angle_k, w_dih_k, b_k):
    vec_r = vec.reshape(B * N * 3, H)
    x_r = x_scalar.reshape(B * N, H)
    zrow = jnp.zeros_like(b_q)
    w_all = jnp.stack(
        [w_angle_q, w_dih_q, b_q, w_angle_k, w_dih_k, b_k, zrow, zrow])

    mesh = plsc.VectorSubcoreMesh(core_axis_name="c", subcore_axis_name="s")
    run = pl.kernel(
        _sc_body,
        out_type=(
            jax.ShapeDtypeStruct((B * N, H), jnp.float32),
            jax.ShapeDtypeStruct((B * N, H), jnp.float32),
        ),
        mesh=mesh,
        scratch_types=[
            pltpu.VMEM((3 * VROWS, H), jnp.float32),
            pltpu.VMEM((XROWS, H), jnp.float32),
            pltpu.VMEM((3 * UROWS, H), jnp.float32),
            pltpu.VMEM((C, H), jnp.float32),
            pltpu.VMEM((C, H), jnp.float32),
            pltpu.VMEM((8, H), jnp.float32),
        ],
    )
    q_r, k_r = run(vec_r, x_r, w_all)
    return q_r.reshape(B, N, H), k_r.reshape(B, N, H)


# speedup vs baseline: 15.8677x; 1.1508x over previous
import functools

import jax
import jax.numpy as jnp
from jax import lax
from jax.experimental import pallas as pl
from jax.experimental.pallas import tpu as pltpu
from jax.experimental.pallas import tpu_sc as plsc

B, N, H, W = 2, 2048, 128, 3
EPS = 1e-8
EPS2 = EPS * EPS
C = 64
NW = 32
LANES = 16
NCG = H // LANES
OFFS = (-3, -2, -1, 1, 2, 3)
VROWS = C + 16
UROWS = C + 6
XROWS = C + 16


def _rsqrt_nr(s):
    i = lax.bitcast_convert_type(s, jnp.int32)
    y = lax.bitcast_convert_type(jnp.int32(0x5F3759DF) - (i >> 1), jnp.float32)
    for _ in range(2):
        y = y * (1.5 - 0.5 * s * y * y)
    return y


def _inv_norm(s):
    return _rsqrt_nr(jnp.maximum(s, EPS2))


def _sc_body(vec_hbm, x_hbm, w_hbm, q_hbm, k_hbm, vecl, xl, ul, ql, kl, wl):
    wid = lax.axis_index("s") * 2 + lax.axis_index("c")
    n0 = wid * C
    sv = jnp.clip(n0 - 8, 0, N - VROWS)
    sx = jnp.clip(n0 - 8, 0, N - XROWS)

    pltpu.sync_copy(w_hbm, wl)

    def chunk_body(it, _):
        bb = it * N
        pltpu.sync_copy(
            vec_hbm.at[pl.ds(pl.multiple_of(3 * (bb + sv), 8), 3 * VROWS)],
            vecl)
        pltpu.sync_copy(
            x_hbm.at[pl.ds(pl.multiple_of(bb + sx, 8), XROWS)], xl)

        def phase_a(ii, _):
            n = n0 - 3 + ii
            r = jnp.clip(n - sv, 0, VROWS - 1)
            vi_ok = jnp.where((n >= 0) & (n < N), 1.0, 0.0)

            def ch_a(c, _):
                cs = c * LANES
                sl = pl.ds(cs, LANES)
                vix = vecl[3 * r, sl]
                viy = vecl[3 * r + 1, sl]
                viz = vecl[3 * r + 2, sl]
                ux = jnp.zeros((LANES,), jnp.float32)
                uy = jnp.zeros((LANES,), jnp.float32)
                uz = jnp.zeros((LANES,), jnp.float32)
                for o in OFFS:
                    n2 = n + o
                    r2 = jnp.clip(n2 - sv, 0, VROWS - 1)
                    bx = vecl[3 * r2, sl] - vix
                    by = vecl[3 * r2 + 1, sl] - viy
                    bz = vecl[3 * r2 + 2, sl] - viz
                    s = bx * bx + by * by + bz * bz
                    ok = vi_ok * jnp.where((n2 >= 0) & (n2 < N), 1.0, 0.0)
                    f = _inv_norm(s) * ok
                    ux = ux + bx * f
                    uy = uy + by * f
                    uz = uz + bz * f
                ul[3 * ii, sl] = ux
                ul[3 * ii + 1, sl] = uy
                ul[3 * ii + 2, sl] = uz
                return 0

            lax.fori_loop(0, NCG, ch_a, 0, unroll=False)
            return 0

        lax.fori_loop(0, UROWS, phase_a, 0, unroll=False)

        def phase_b(c, _):
            cs = c * LANES
            sl = pl.ds(cs, LANES)
            w0q = wl[0, sl]
            w1q = wl[1, sl]
            w2q = wl[2, sl]
            w0k = wl[3, sl]
            w1k = wl[4, sl]
            w2k = wl[5, sl]

            def node_b(i, _):
                n = n0 + i
                r = n - sv
                vix = vecl[3 * r, sl]
                viy = vecl[3 * r + 1, sl]
                viz = vecl[3 * r + 2, sl]
                ur = i + 3
                uix = ul[3 * ur, sl]
                uiy = ul[3 * ur + 1, sl]
                uiz = ul[3 * ur + 2, sl]
                s_ui = uix * uix + uiy * uiy + uiz * uiz
                inv_ui = _inv_norm(s_ui)
                q_acc = jnp.zeros((LANES,), jnp.float32)
                k_acc = jnp.zeros((LANES,), jnp.float32)
                for o in OFFS:
                    n2 = n + o
                    ok = jnp.where((n2 >= 0) & (n2 < N), 1.0, 0.0)
                    r2 = jnp.clip(n2 - sv, 0, VROWS - 1)
                    bx = vecl[3 * r2, sl] - vix
                    by = vecl[3 * r2 + 1, sl] - viy
                    bz = vecl[3 * r2 + 2, sl] - viz
                    s_e = bx * bx + by * by + bz * bz
                    inv_e = _inv_norm(s_e)
                    ex = bx * inv_e
                    ey = by * inv_e
                    ez = bz * inv_e
                    u2 = ur + o
                    ujx = ul[3 * u2, sl]
                    ujy = ul[3 * u2 + 1, sl]
                    ujz = ul[3 * u2 + 2, sl]
                    d_i = uix * ex + uiy * ey + uiz * ez
                    d_j = ujx * ex + ujy * ey + ujz * ez
                    ang = jnp.maximum(jnp.minimum(d_i * inv_ui, 1.0), -1.0)
                    uipx = uix - d_i * ex
                    uipy = uiy - d_i * ey
                    uipz = uiz - d_i * ez
                    ujpx = ujx - d_j * ex
                    ujpy = ujy - d_j * ey
                    ujpz = ujz - d_j * ez
                    s_pi = uipx * uipx + uipy * uipy + uipz * uipz
                    s_pj = ujpx * ujpx + ujpy * ujpy + ujpz * ujpz
                    dotp = uipx * ujpx + uipy * ujpy + uipz * ujpz
                    spp = jnp.maximum(s_pi, EPS2) * jnp.maximum(s_pj, EPS2)
                    dih = dotp * _rsqrt_nr(spp)
                    dih = jnp.maximum(jnp.minimum(dih, 1.0), -1.0)
                    zq = w2q + ang * w0q + dih * w1q
                    zk = w2k + ang * w0k + dih * w1k
                    gq = 1.0 / (1.0 + jnp.exp(-zq))
                    gk = 1.0 / (1.0 + jnp.exp(-zk))
                    rx = jnp.clip(n2 - sx, 0, XROWS - 1)
                    xj = xl[rx, sl] * ok
                    q_acc = q_acc + gq * xj
                    k_acc = k_acc + gk * xj
                ql[i, sl] = q_acc
                kl[i, sl] = k_acc
                return 0

            lax.fori_loop(0, C, node_b, 0, unroll=False)
            return 0

        lax.fori_loop(0, NCG, phase_b, 0, unroll=False)

        pltpu.sync_copy(ql, q_hbm.at[pl.ds(pl.multiple_of(bb + n0, 8), C)])
        pltpu.sync_copy(kl, k_hbm.at[pl.ds(pl.multiple_of(bb + n0, 8), C)])
        return 0

    lax.fori_loop(0, B, chunk_body, 0, unroll=False)


@jax.jit
def kernel(x_scalar, vec, w_angle_q, w_dih_q, b_q, w_angle_k, w_dih_k, b_k):
    vec_r = vec.reshape(B * N * 3, H)
    x_r = x_scalar.reshape(B * N, H)
    zrow = jnp.zeros_like(b_q)
    w_all = jnp.stack(
        [w_angle_q, w_dih_q, b_q, w_angle_k, w_dih_k, b_k, zrow, zrow])

    mesh = plsc.VectorSubcoreMesh(core_axis_name="c", subcore_axis_name="s")
    run = pl.kernel(
        _sc_body,
        out_type=(
            jax.ShapeDtypeStruct((B * N, H), jnp.float32),
            jax.ShapeDtypeStruct((B * N, H), jnp.float32),
        ),
        mesh=mesh,
        scratch_types=[
            pltpu.VMEM((3 * VROWS, H), jnp.float32),
            pltpu.VMEM((XROWS, H), jnp.float32),
            pltpu.VMEM((3 * UROWS, H), jnp.float32),
            pltpu.VMEM((C, H), jnp.float32),
            pltpu.VMEM((C, H), jnp.float32),
            pltpu.VMEM((8, H), jnp.float32),
        ],
    )
    q_r, k_r = run(vec_r, x_r, w_all)
    return q_r.reshape(B, N, H), k_r.reshape(B, N, H)
